# Initial kernel scaffold; baseline (speedup 1.0000x reference)
import jax
import jax.numpy as jnp
from jax.experimental import pallas as pl


def kernel(x, edge_index, W1, att_src1, att_dst1, b1, W2, att_src2, att_dst2, b2):
    raise NotImplementedError("write your pallas kernel here")


# speedup vs baseline: 21.9032x; 21.9032x over previous
import functools

import jax
import jax.numpy as jnp
from jax import lax
from jax.experimental import pallas as pl
from jax.experimental.pallas import tpu as pltpu
from jax.experimental.pallas import tpu_sc as plsc

N = 50000
E = 800000
F = 1433
D1 = 64
H1 = 8
D2 = 16
C2 = 7

NSUB = 16
NCORE = 2
CHUNK = 512
SUBC = 128
EP = 802816
CH_PER_SUB = EP // (CHUNK * NSUB)
EROWS = EP // SUBC
HALF = 25000
QN = 12544
SPQ = 12672
NBLKQ = SPQ // 128
NBLKO = QN // 128
SP_ROWS = 25088
NBLK = SP_ROWS // 128
NPAD = 50008


def _k1(x, W1, As, Ad):
    BN = 1024
    grid = (pl.cdiv(N, BN),)

    def body(x_ref, w_ref, as_ref, ad_ref, h_ref, s_ref, d_ref):
        h = jnp.dot(x_ref[...], w_ref[...], preferred_element_type=jnp.float32)
        h_ref[...] = h
        s_ref[...] = jnp.dot(h, as_ref[...], preferred_element_type=jnp.float32)
        d_ref[...] = jnp.dot(h, ad_ref[...], preferred_element_type=jnp.float32)

    return pl.pallas_call(
        body,
        grid=grid,
        in_specs=[
            pl.BlockSpec((BN, F), lambda i: (i, 0)),
            pl.BlockSpec((F, D1), lambda i: (0, 0)),
            pl.BlockSpec((D1, H1), lambda i: (0, 0)),
            pl.BlockSpec((D1, H1), lambda i: (0, 0)),
        ],
        out_specs=[
            pl.BlockSpec((BN, D1), lambda i: (i, 0)),
            pl.BlockSpec((BN, H1), lambda i: (i, 0)),
            pl.BlockSpec((BN, H1), lambda i: (i, 0)),
        ],
        out_shape=[
            jax.ShapeDtypeStruct((N, D1), jnp.float32),
            jax.ShapeDtypeStruct((N, H1), jnp.float32),
            jax.ShapeDtypeStruct((N, H1), jnp.float32),
        ],
    )(x, W1, As, Ad)


def _k2_body(src2, dst2, h1t, asrt, adrt, cvech, z64, z8,
             acc_o, den_o,
             src_v, dst_v, dl_v, rows_v, asr_v, adr_v, aw_v,
             zb64, zb8, cvec_v, acc_sh, den_sh, sem):
    cid = lax.axis_index("c")
    sid = lax.axis_index("s")
    iot = lax.iota(jnp.int32, 16)
    t01 = iot // 8
    colp = iot % 8

    pltpu.sync_copy(z64, zb64)
    pltpu.sync_copy(z8, zb8)
    pltpu.sync_copy(cvech, cvec_v)
    cv = cvec_v[...]

    for p in range(2):
        qid = cid * 2 + p
        lo = qid * QN

        def zinit(t, c):
            b = t * NSUB + sid

            @pl.when(b < NBLKQ)
            def _():
                pltpu.sync_copy(zb64, acc_sh.at[pl.ds(b * 128, 128)])
                pltpu.sync_copy(zb8, den_sh.at[pl.ds(b * 128, 128)])
            return c

        lax.fori_loop(0, (NBLKQ + NSUB - 1) // NSUB, zinit, 0)
        plsc.subcore_barrier()

        def chunk(i, c):
            row0 = sid * (CH_PER_SUB * 4) + i * 4
            pltpu.sync_copy(src2.at[pl.ds(row0, 4)], src_v)
            pltpu.sync_copy(dst2.at[pl.ds(row0, 4)], dst_v)
            descs = []
            for q in range(4):
                descs.append(pltpu.async_copy(
                    h1t.at[src_v.at[q]], rows_v.at[pl.ds(q * SUBC, SUBC)], sem))
                descs.append(pltpu.async_copy(
                    asrt.at[src_v.at[q]], asr_v.at[pl.ds(q * SUBC, SUBC)], sem))
                descs.append(pltpu.async_copy(
                    adrt.at[dst_v.at[q]], adr_v.at[pl.ds(q * SUBC, SUBC)], sem))
            for d in descs:
                d.wait()

            def dlk(j, c2):
                q = j // 8
                r = j % 8
                qv = jnp.full((16,), q, jnp.int32)
                cvi = r * 16 + iot
                d = plsc.load_gather(dst_v, [qv, cvi])
                ok = (d >= lo) & (d < lo + QN)
                dl = jnp.where(ok, d - lo, QN)
                plsc.store_scatter(dl_v, [qv, cvi], dl)
                return c2

            lax.fori_loop(0, 32, dlk, 0)

            def awk(k, c2):
                rp = 2 * k + t01
                sv = (plsc.load_gather(asr_v, [rp, colp]) +
                      plsc.load_gather(adr_v, [rp, colp]))
                sv = jnp.where(sv > 0, sv, 0.2 * sv)
                w = jnp.exp(sv - cv)
                plsc.store_scatter(aw_v, [rp, colp], w)
                return c2

            lax.fori_loop(0, 256, awk, 0)

            def me(e, c2):
                ev = jnp.full((16,), e, jnp.int32)
                for j in range(4):
                    m = plsc.load_gather(aw_v, [ev, 2 * j + t01])
                    r = plsc.load_gather(rows_v, [ev, 16 * j + iot])
                    plsc.store_scatter(rows_v, [ev, 16 * j + iot], r * m)
                return c2

            lax.fori_loop(0, CHUNK, me, 0)

            for q in range(4):
                pltpu.sync_copy(rows_v.at[pl.ds(q * SUBC, SUBC)],
                                acc_sh.at[dl_v.at[q]], add=True)
                pltpu.sync_copy(aw_v.at[pl.ds(q * SUBC, SUBC)],
                                den_sh.at[dl_v.at[q]], add=True)
            return c

        lax.fori_loop(0, CH_PER_SUB, chunk, 0)
        plsc.subcore_barrier()

        def cout(t, c):
            b = t * NSUB + sid

            @pl.when(b < NBLKO)
            def _():
                pltpu.sync_copy(acc_sh.at[pl.ds(b * 128, 128)], zb64)
                pltpu.sync_copy(zb64, acc_o.at[qid].at[pl.ds(b * 128, 128)])
                pltpu.sync_copy(den_sh.at[pl.ds(b * 128, 128)], zb8)
                pltpu.sync_copy(zb8, den_o.at[qid].at[pl.ds(b * 128, 128)])
            return c

        lax.fori_loop(0, (NBLKO + NSUB - 1) // NSUB, cout, 0)
        plsc.subcore_barrier()


def _k2(src2, dst2, h1t, asrt, adrt, cvech, z64, z8):
    mesh = plsc.VectorSubcoreMesh(core_axis_name="c", subcore_axis_name="s")
    fn = pl.kernel(
        _k2_body,
        out_type=[
            jax.ShapeDtypeStruct((4, QN, D1), jnp.float32),
            jax.ShapeDtypeStruct((4, QN, H1), jnp.float32),
        ],
        mesh=mesh,
        compiler_params=pltpu.CompilerParams(
            use_tc_tiling_on_sc=False, needs_layout_passes=False),
        scratch_types=[
            pltpu.VMEM((4, SUBC), jnp.int32),
            pltpu.VMEM((4, SUBC), jnp.int32),
            pltpu.VMEM((4, SUBC), jnp.int32),
            pltpu.VMEM((CHUNK, D1), jnp.float32),
            pltpu.VMEM((CHUNK, H1), jnp.float32),
            pltpu.VMEM((CHUNK, H1), jnp.float32),
            pltpu.VMEM((CHUNK, H1), jnp.float32),
            pltpu.VMEM((128, D1), jnp.float32),
            pltpu.VMEM((128, H1), jnp.float32),
            pltpu.VMEM((16,), jnp.float32),
            pltpu.VMEM_SHARED((SPQ, D1), jnp.float32),
            pltpu.VMEM_SHARED((SPQ, H1), jnp.float32),
            pltpu.SemaphoreType.DMA,
        ],
    )
    return fn(src2, dst2, h1t, asrt, adrt, cvech, z64, z8)


def _k3(acc, den, b1, w2p, qrow, rrep):
    BN = 1024
    grid = (pl.cdiv(N, BN),)

    def body(a_ref, dn_ref, b1_ref, w_ref, q_ref, r_ref, o_ref):
        a = a_ref[...]
        dn64 = jnp.dot(dn_ref[...], r_ref[...],
                       preferred_element_type=jnp.float32)
        o = a / (dn64 + 1e-16) + b1_ref[...]
        el = jnp.where(o > 0, o, jnp.exp(o) - 1.0)
        o_ref[...] = (jnp.dot(el, w_ref[...],
                              preferred_element_type=jnp.float32) + q_ref[...])

    return pl.pallas_call(
        body,
        grid=grid,
        in_specs=[
            pl.BlockSpec((BN, D1), lambda i: (i, 0)),
            pl.BlockSpec((BN, H1), lambda i: (i, 0)),
            pl.BlockSpec((1, D1), lambda i: (0, 0)),
            pl.BlockSpec((D1, D2), lambda i: (0, 0)),
            pl.BlockSpec((1, D2), lambda i: (0, 0)),
            pl.BlockSpec((H1, D1), lambda i: (0, 0)),
        ],
        out_specs=pl.BlockSpec((BN, D2), lambda i: (i, 0)),
        out_shape=jax.ShapeDtypeStruct((N, D2), jnp.float32),
    )(acc, den, b1, w2p, qrow, rrep)


def _k4_body(src2, dst2, h2t, adt2, cvech, z16,
             acc_o,
             src_v, dst_v, dl_v, rows_v, adv_v, aw_v,
             zb16, cvec_v, acc_sh, sem):
    cid = lax.axis_index("c")
    sid = lax.axis_index("s")
    lo = cid * HALF
    iot = lax.iota(jnp.int32, 16)

    pltpu.sync_copy(z16, zb16)
    pltpu.sync_copy(cvech, cvec_v)

    def zinit(t, c):
        b = t * NSUB + sid

        @pl.when(b < NBLK)
        def _():
            pltpu.sync_copy(zb16, acc_sh.at[pl.ds(b * 128, 128)])
        return c

    lax.fori_loop(0, (NBLK + NSUB - 1) // NSUB, zinit, 0)
    plsc.subcore_barrier()

    cv = cvec_v[...]

    def chunk(i, c):
        row0 = sid * (CH_PER_SUB * 4) + i * 4
        pltpu.sync_copy(src2.at[pl.ds(row0, 4)], src_v)
        pltpu.sync_copy(dst2.at[pl.ds(row0, 4)], dst_v)
        descs = []
        for q in range(4):
            descs.append(pltpu.async_copy(
                h2t.at[src_v.at[q]], rows_v.at[pl.ds(q * SUBC, SUBC)], sem))
            descs.append(pltpu.async_copy(
                adt2.at[dst_v.at[q]], adv_v.at[q], sem))
        for d in descs:
            d.wait()

        def dlk(j, c2):
            q = j // 8
            r = j % 8
            qv = jnp.full((16,), q, jnp.int32)
            cvi = r * 16 + iot
            d = plsc.load_gather(dst_v, [qv, cvi])
            ok = (d >= lo) & (d < lo + HALF)
            dl = jnp.where(ok, d - lo, HALF)
            plsc.store_scatter(dl_v, [qv, cvi], dl)
            return c2

        lax.fori_loop(0, 32, dlk, 0)

        def awk(k, c2):
            q = k // 8
            r = k % 8
            qv = jnp.full((16,), q, jnp.int32)
            s = plsc.load_gather(rows_v, [k * 16 + iot, jnp.full((16,), 8, jnp.int32)])
            a = plsc.load_gather(adv_v, [qv, r * 16 + iot])
            s = s + a
            s = jnp.where(s > 0, s, 0.2 * s)
            w = jnp.exp(s - cv)
            plsc.store_scatter(aw_v, [k * 16 + iot], w)
            return c2

        lax.fori_loop(0, 32, awk, 0)

        def me(e, c2):
            ev = jnp.full((16,), e, jnp.int32)
            m = plsc.load_gather(aw_v, [ev])
            r = plsc.load_gather(rows_v, [ev, iot])
            plsc.store_scatter(rows_v, [ev, iot], r * m)
            return c2

        lax.fori_loop(0, CHUNK, me, 0)

        for q in range(4):
            pltpu.sync_copy(rows_v.at[pl.ds(q * SUBC, SUBC)],
                            acc_sh.at[dl_v.at[q]], add=True)
        return c

    lax.fori_loop(0, CH_PER_SUB, chunk, 0)
    plsc.subcore_barrier()

    def cout(t, c):
        b = t * NSUB + sid

        @pl.when(b < NBLK)
        def _():
            pltpu.sync_copy(acc_sh.at[pl.ds(b * 128, 128)], zb16)
            pltpu.sync_copy(zb16, acc_o.at[cid].at[pl.ds(b * 128, 128)])
        return c

    lax.fori_loop(0, (NBLK + NSUB - 1) // NSUB, cout, 0)


def _k4(src2, dst2, h2t, adt2, cvech, z16):
    mesh = plsc.VectorSubcoreMesh(core_axis_name="c", subcore_axis_name="s")
    fn = pl.kernel(
        _k4_body,
        out_type=[
            jax.ShapeDtypeStruct((NCORE, SP_ROWS, D2), jnp.float32),
        ],
        mesh=mesh,
        compiler_params=pltpu.CompilerParams(
            use_tc_tiling_on_sc=False, needs_layout_passes=False),
        scratch_types=[
            pltpu.VMEM((4, SUBC), jnp.int32),
            pltpu.VMEM((4, SUBC), jnp.int32),
            pltpu.VMEM((4, SUBC), jnp.int32),
            pltpu.VMEM((CHUNK, D2), jnp.float32),
            pltpu.VMEM((4, SUBC), jnp.float32),
            pltpu.VMEM((CHUNK,), jnp.float32),
            pltpu.VMEM((128, D2), jnp.float32),
            pltpu.VMEM((16,), jnp.float32),
            pltpu.VMEM_SHARED((SP_ROWS, D2), jnp.float32),
            pltpu.SemaphoreType.DMA,
        ],
    )
    return fn(src2, dst2, h2t, adt2, cvech, z16)


def _k5(acc2, b2):
    BN = 1024
    grid = (pl.cdiv(N, BN),)

    def body(a_ref, b2_ref, o_ref):
        a = a_ref[...]
        v = a[:, 0:C2] / (a[:, C2:C2 + 1] + 1e-16) + b2_ref[...]
        m = jnp.max(v, axis=-1, keepdims=True)
        vs = v - m
        o_ref[...] = vs - jnp.log(jnp.sum(jnp.exp(vs), axis=-1, keepdims=True))

    return pl.pallas_call(
        body,
        grid=grid,
        in_specs=[
            pl.BlockSpec((BN, D2), lambda i: (i, 0)),
            pl.BlockSpec((1, C2), lambda i: (0, 0)),
        ],
        out_specs=pl.BlockSpec((BN, C2), lambda i: (i, 0)),
        out_shape=jax.ShapeDtypeStruct((N, C2), jnp.float32),
    )(acc2, b2)


def _lrelu(v):
    return jnp.where(v > 0, v, 0.2 * v)


@jax.jit
def _run(x, edge_index, W1, att_src1, att_dst1, b1, W2, att_src2, att_dst2, b2):
    xs = x[0]
    ei = edge_index[0].astype(jnp.int32)
    src = jnp.concatenate([ei[0], jnp.zeros((EP - E,), jnp.int32)])
    dst = jnp.concatenate([ei[1], jnp.full((EP - E,), N, jnp.int32)])
    src2 = src.reshape(EROWS, SUBC)
    dst2 = dst.reshape(EROWS, SUBC)

    eye8 = jnp.eye(H1, dtype=jnp.float32)
    As = (att_src1[0][:, :, None] * eye8[:, None, :]).reshape(D1, H1)
    Ad = (att_dst1[0][:, :, None] * eye8[:, None, :]).reshape(D1, H1)

    h1, asr, adr = _k1(xs, W1, As, Ad)

    c1 = _lrelu(jnp.max(asr, axis=0) + jnp.max(adr, axis=0))
    cvec1 = jnp.tile(c1, 2)
    adrt = jnp.concatenate([adr, jnp.zeros((NPAD - N, H1), jnp.float32)])

    z64 = jnp.zeros((128, D1), jnp.float32)
    z8 = jnp.zeros((128, H1), jnp.float32)
    accs, dens = _k2(src2, dst2, h1, asr, adrt, cvec1, z64, z8)

    acc = accs.reshape(4 * QN, D1)[:N]
    den = dens.reshape(4 * QN, H1)[:N]

    a_s = att_src2[0, 0]
    a_d = att_dst2[0, 0]
    w2p = jnp.zeros((D1, D2), jnp.float32)
    w2p = w2p.at[:, 0:C2].set(W2)
    w2p = w2p.at[:, 8].set(W2 @ a_s)
    w2p = w2p.at[:, 9].set(W2 @ a_d)
    qrow = jnp.zeros((1, D2), jnp.float32).at[0, C2].set(1.0)
    rrep = (eye8[:, :, None] * jnp.ones((1, 1, 8))).reshape(H1, D1)

    h2t = _k3(acc, den, b1.reshape(1, D1), w2p, qrow, rrep)

    c2 = _lrelu(jnp.max(h2t[:, 8]) + jnp.max(h2t[:, 9]))
    cvec2 = jnp.full((16,), c2, jnp.float32)
    adt2 = jnp.concatenate([h2t[:, 9], jnp.zeros((NPAD - N,), jnp.float32)])
    z16 = jnp.zeros((128, D2), jnp.float32)

    (acc2s,) = _k4(src2, dst2, h2t, adt2, cvec2, z16)
    acc2 = jnp.concatenate([acc2s[0, :HALF], acc2s[1, :HALF]])

    out = _k5(acc2, b2.reshape(1, C2))
    return out.reshape(1, N, C2)


def kernel(x, edge_index, W1, att_src1, att_dst1, b1, W2, att_src2, att_dst2, b2):
    return _run(x, edge_index, W1, att_src1, att_dst1, b1,
                W2, att_src2, att_dst2, b2)


# speedup vs baseline: 32.9495x; 1.5043x over previous
import jax
import jax.numpy as jnp
from jax import lax
from jax.experimental import pallas as pl
from jax.experimental.pallas import tpu as pltpu
from jax.experimental.pallas import tpu_sc as plsc

N = 50000
E = 800000
F = 1433
D1 = 64
H1 = 8
D2 = 16
C2 = 7

NSUB = 16
CHUNK = 128
SUBC = 128
SUBQ = CHUNK // SUBC
NRING = 4
EP = 802816
CH_PER_SUB = EP // (CHUNK * NSUB)
EROWS = EP // SUBC
HALF = 25000
QN = 12544
SPQ = 12672
NBLKQ = SPQ // 128
NBLKO = QN // 128
SP_ROWS = 25088
NBLK = SP_ROWS // 128
NPAD = 50008
N3 = 4 * QN


def _k1(x, W1, As, Ad):
    BN = 1024
    grid = (pl.cdiv(N, BN),)

    def body(x_ref, w_ref, as_ref, ad_ref, h_ref, s_ref, d_ref):
        h = jnp.dot(x_ref[...], w_ref[...], preferred_element_type=jnp.float32)
        h_ref[...] = h
        s_ref[...] = jnp.dot(h, as_ref[...], preferred_element_type=jnp.float32)
        d_ref[...] = jnp.dot(h, ad_ref[...], preferred_element_type=jnp.float32)

    return pl.pallas_call(
        body,
        grid=grid,
        in_specs=[
            pl.BlockSpec((BN, F), lambda i: (i, 0)),
            pl.BlockSpec((F, D1), lambda i: (0, 0)),
            pl.BlockSpec((D1, H1), lambda i: (0, 0)),
            pl.BlockSpec((D1, H1), lambda i: (0, 0)),
        ],
        out_specs=[
            pl.BlockSpec((BN, D1), lambda i: (i, 0)),
            pl.BlockSpec((BN, H1), lambda i: (i, 0)),
            pl.BlockSpec((BN, H1), lambda i: (i, 0)),
        ],
        out_shape=[
            jax.ShapeDtypeStruct((N, D1), jnp.float32),
            jax.ShapeDtypeStruct((N, H1), jnp.float32),
            jax.ShapeDtypeStruct((N, H1), jnp.float32),
        ],
    )(x, W1, As, Ad)


def _k2_body(src2, dst2, h1t, asrt, adrt, cvech, z64, z8,
             acc_o, den_o,
             src_v, dst_v, dl_v, rows_v, asr_v, adr_v, aw_v,
             zb64, zb8, cvec_v, acc_sh, den_sh,
             g0, g1, g2, g3, s0, s1, s2, s3):
    semg = (g0, g1, g2, g3)
    sems = (s0, s1, s2, s3)
    cid = lax.axis_index("c")
    sid = lax.axis_index("s")
    iot = lax.iota(jnp.int32, 16)
    t01 = iot // 8
    colp = iot % 8

    pltpu.sync_copy(z64, zb64)
    pltpu.sync_copy(z8, zb8)
    pltpu.sync_copy(cvech, cvec_v)
    cv = cvec_v[...]

    def row0_of(c):
        return sid * (CH_PER_SUB * SUBQ) + c * SUBQ

    def fire_g(c, t):
        r0 = row0_of(c)
        pltpu.sync_copy(src2.at[pl.ds(r0, SUBQ)], src_v.at[t])
        pltpu.sync_copy(dst2.at[pl.ds(r0, SUBQ)], dst_v.at[t])
        for q in range(SUBQ):
            pltpu.async_copy(h1t.at[src_v.at[t].at[q]],
                             rows_v.at[t].at[pl.ds(q * SUBC, SUBC)], semg[t])
            pltpu.async_copy(asrt.at[src_v.at[t].at[q]],
                             asr_v.at[t].at[pl.ds(q * SUBC, SUBC)], semg[t])
            pltpu.async_copy(adrt.at[dst_v.at[t].at[q]],
                             adr_v.at[t].at[pl.ds(q * SUBC, SUBC)], semg[t])

    def drain_g(t):
        for q in range(SUBQ):
            pltpu.make_async_copy(
                h1t.at[src_v.at[t].at[q]],
                rows_v.at[t].at[pl.ds(q * SUBC, SUBC)], semg[t]).wait()
            pltpu.make_async_copy(
                asrt.at[src_v.at[t].at[q]],
                asr_v.at[t].at[pl.ds(q * SUBC, SUBC)], semg[t]).wait()
            pltpu.make_async_copy(
                adrt.at[dst_v.at[t].at[q]],
                adr_v.at[t].at[pl.ds(q * SUBC, SUBC)], semg[t]).wait()

    def fire_s(t):
        for q in range(SUBQ):
            pltpu.async_copy(rows_v.at[t].at[pl.ds(q * SUBC, SUBC)],
                             acc_sh.at[dl_v.at[t].at[q]], sems[t], add=True)
            pltpu.async_copy(aw_v.at[t].at[pl.ds(q * SUBC, SUBC)],
                             den_sh.at[dl_v.at[t].at[q]], sems[t], add=True)

    def drain_s(t):
        for q in range(SUBQ):
            pltpu.make_async_copy(rows_v.at[t].at[pl.ds(q * SUBC, SUBC)],
                                  acc_sh.at[dl_v.at[t].at[q]], sems[t]).wait()
            pltpu.make_async_copy(aw_v.at[t].at[pl.ds(q * SUBC, SUBC)],
                                  den_sh.at[dl_v.at[t].at[q]], sems[t]).wait()

    def compute(lo, t):
        @plsc.parallel_loop(0, SUBQ * 8, unroll=2)
        def _dl(j):
            q = j // 8
            r = j % 8
            qv = jnp.full((16,), q, jnp.int32)
            cvi = r * 16 + iot
            d = plsc.load_gather(dst_v.at[t], [qv, cvi])
            ok = (d >= lo) & (d < lo + QN)
            dl = jnp.where(ok, d - lo, QN)
            plsc.store_scatter(dl_v.at[t], [qv, cvi], dl)

        @plsc.parallel_loop(0, CHUNK // 2, unroll=2)
        def _aw(k):
            rp = 2 * k + t01
            sv = (plsc.load_gather(asr_v.at[t], [rp, colp]) +
                  plsc.load_gather(adr_v.at[t], [rp, colp]))
            sv = jnp.where(sv > 0, sv, 0.2 * sv)
            w = jnp.exp(sv - cv)
            plsc.store_scatter(aw_v.at[t], [rp, colp], w)

        @plsc.parallel_loop(0, CHUNK, unroll=2)
        def _me(e):
            ev = jnp.full((16,), e, jnp.int32)
            for j in range(4):
                m = plsc.load_gather(aw_v.at[t], [ev, 2 * j + t01])
                r = plsc.load_gather(rows_v.at[t], [ev, 16 * j + iot])
                plsc.store_scatter(rows_v.at[t], [ev, 16 * j + iot], r * m)

    for p in range(2):
        qid = cid * 2 + p
        lo = qid * QN

        def zinit(tt, c):
            b = tt * NSUB + sid

            @pl.when(b < NBLKQ)
            def _():
                pltpu.sync_copy(zb64, acc_sh.at[pl.ds(b * 128, 128)])
                pltpu.sync_copy(zb8, den_sh.at[pl.ds(b * 128, 128)])
            return c

        lax.fori_loop(0, (NBLKQ + NSUB - 1) // NSUB, zinit, 0)
        fire_g(0, 0)
        plsc.subcore_barrier()

        def outer(i, c_):
            for b in range(NRING):
                cthis = i * NRING + b
                b2 = (b - 2) % NRING
                if b >= 2:
                    drain_s(b2)
                else:
                    @pl.when(i >= 1)
                    def _():
                        drain_s(b2)
                if b == NRING - 1:
                    @pl.when(cthis + 1 < CH_PER_SUB)
                    def _():
                        fire_g(cthis + 1, 0)
                else:
                    fire_g(cthis + 1, b + 1)
                drain_g(b)
                compute(lo, b)
                fire_s(b)
            return c_

        lax.fori_loop(0, CH_PER_SUB // NRING, outer, 0)
        drain_s(2)
        drain_s(3)
        plsc.subcore_barrier()

        def cout(tt, c):
            b = tt * NSUB + sid

            @pl.when(b < NBLKO)
            def _():
                pltpu.sync_copy(acc_sh.at[pl.ds(b * 128, 128)], zb64)
                pltpu.sync_copy(zb64, acc_o.at[qid].at[pl.ds(b * 128, 128)])
                pltpu.sync_copy(den_sh.at[pl.ds(b * 128, 128)], zb8)
                pltpu.sync_copy(zb8, den_o.at[qid].at[pl.ds(b * 128, 128)])
            return c

        lax.fori_loop(0, (NBLKO + NSUB - 1) // NSUB, cout, 0)
        plsc.subcore_barrier()


def _k2(src2, dst2, h1t, asrt, adrt, cvech, z64, z8):
    mesh = plsc.VectorSubcoreMesh(core_axis_name="c", subcore_axis_name="s")
    fn = pl.kernel(
        _k2_body,
        out_type=[
            jax.ShapeDtypeStruct((4, QN, D1), jnp.float32),
            jax.ShapeDtypeStruct((4, QN, H1), jnp.float32),
        ],
        mesh=mesh,
        compiler_params=pltpu.CompilerParams(
            use_tc_tiling_on_sc=False, needs_layout_passes=False),
        scratch_types=[
            pltpu.VMEM((NRING, SUBQ, SUBC), jnp.int32),
            pltpu.VMEM((NRING, SUBQ, SUBC), jnp.int32),
            pltpu.VMEM((NRING, SUBQ, SUBC), jnp.int32),
            pltpu.VMEM((NRING, CHUNK, D1), jnp.float32),
            pltpu.VMEM((NRING, CHUNK, H1), jnp.float32),
            pltpu.VMEM((NRING, CHUNK, H1), jnp.float32),
            pltpu.VMEM((NRING, CHUNK, H1), jnp.float32),
            pltpu.VMEM((128, D1), jnp.float32),
            pltpu.VMEM((128, H1), jnp.float32),
            pltpu.VMEM((16,), jnp.float32),
            pltpu.VMEM_SHARED((SPQ, D1), jnp.float32),
            pltpu.VMEM_SHARED((SPQ, H1), jnp.float32),
            pltpu.SemaphoreType.DMA,
            pltpu.SemaphoreType.DMA,
            pltpu.SemaphoreType.DMA,
            pltpu.SemaphoreType.DMA,
            pltpu.SemaphoreType.DMA,
            pltpu.SemaphoreType.DMA,
            pltpu.SemaphoreType.DMA,
            pltpu.SemaphoreType.DMA,
        ],
    )
    return fn(src2, dst2, h1t, asrt, adrt, cvech, z64, z8)


def _k3(acc, den, b1, w2p, qrow, rrep):
    BN = 1024
    grid = (N3 // BN,)

    def body(a_ref, dn_ref, b1_ref, w_ref, q_ref, r_ref,
             o_ref, as_ref, ad_ref):
        a = a_ref[...]
        dn64 = jnp.dot(dn_ref[...], r_ref[...],
                       preferred_element_type=jnp.float32)
        o = a / (dn64 + 1e-16) + b1_ref[...]
        el = jnp.where(o > 0, o, jnp.exp(o) - 1.0)
        h2x = (jnp.dot(el, w_ref[...],
                       preferred_element_type=jnp.float32) + q_ref[...])
        o_ref[...] = h2x
        as_ref[...] = h2x[:, 8].reshape(BN // 128, 128)
        ad_ref[...] = h2x[:, 9].reshape(BN // 128, 128)

    return pl.pallas_call(
        body,
        grid=grid,
        in_specs=[
            pl.BlockSpec((BN, D1), lambda i: (i, 0)),
            pl.BlockSpec((BN, H1), lambda i: (i, 0)),
            pl.BlockSpec((1, D1), lambda i: (0, 0)),
            pl.BlockSpec((D1, D2), lambda i: (0, 0)),
            pl.BlockSpec((1, D2), lambda i: (0, 0)),
            pl.BlockSpec((H1, D1), lambda i: (0, 0)),
        ],
        out_specs=[
            pl.BlockSpec((BN, D2), lambda i: (i, 0)),
            pl.BlockSpec((BN // 128, 128), lambda i: (i, 0)),
            pl.BlockSpec((BN // 128, 128), lambda i: (i, 0)),
        ],
        out_shape=[
            jax.ShapeDtypeStruct((N3, D2), jnp.float32),
            jax.ShapeDtypeStruct((N3 // 128, 128), jnp.float32),
            jax.ShapeDtypeStruct((N3 // 128, 128), jnp.float32),
        ],
    )(acc, den, b1, w2p, qrow, rrep)


def _k4_body(src2, dst2, h2t, adt2, cvech, z16,
             acc_o,
             src_v, dst_v, dl_v, rows_v, adv_v, aw_v,
             zb16, cvec_v, acc_sh,
             g0, g1, g2, g3, s0, s1, s2, s3):
    semg = (g0, g1, g2, g3)
    sems = (s0, s1, s2, s3)
    cid = lax.axis_index("c")
    sid = lax.axis_index("s")
    lo = cid * HALF
    iot = lax.iota(jnp.int32, 16)

    pltpu.sync_copy(z16, zb16)
    pltpu.sync_copy(cvech, cvec_v)
    cv = cvec_v[...]

    def row0_of(c):
        return sid * (CH_PER_SUB * SUBQ) + c * SUBQ

    def fire_g(c, t):
        r0 = row0_of(c)
        pltpu.sync_copy(src2.at[pl.ds(r0, SUBQ)], src_v.at[t])
        pltpu.sync_copy(dst2.at[pl.ds(r0, SUBQ)], dst_v.at[t])
        for q in range(SUBQ):
            pltpu.async_copy(h2t.at[src_v.at[t].at[q]],
                             rows_v.at[t].at[pl.ds(q * SUBC, SUBC)], semg[t])
            pltpu.async_copy(adt2.at[dst_v.at[t].at[q]],
                             adv_v.at[t].at[q], semg[t])

    def drain_g(t):
        for q in range(SUBQ):
            pltpu.make_async_copy(
                h2t.at[src_v.at[t].at[q]],
                rows_v.at[t].at[pl.ds(q * SUBC, SUBC)], semg[t]).wait()
            pltpu.make_async_copy(
                adt2.at[dst_v.at[t].at[q]], adv_v.at[t].at[q], semg[t]).wait()

    def fire_s(t):
        for q in range(SUBQ):
            pltpu.async_copy(rows_v.at[t].at[pl.ds(q * SUBC, SUBC)],
                             acc_sh.at[dl_v.at[t].at[q]], sems[t], add=True)

    def drain_s(t):
        for q in range(SUBQ):
            pltpu.make_async_copy(rows_v.at[t].at[pl.ds(q * SUBC, SUBC)],
                                  acc_sh.at[dl_v.at[t].at[q]], sems[t]).wait()

    def compute(t):
        @plsc.parallel_loop(0, SUBQ * 8, unroll=2)
        def _dl(j):
            q = j // 8
            r = j % 8
            qv = jnp.full((16,), q, jnp.int32)
            cvi = r * 16 + iot
            d = plsc.load_gather(dst_v.at[t], [qv, cvi])
            ok = (d >= lo) & (d < lo + HALF)
            dl = jnp.where(ok, d - lo, HALF)
            plsc.store_scatter(dl_v.at[t], [qv, cvi], dl)

        @plsc.parallel_loop(0, CHUNK // 16, unroll=2)
        def _aw(k):
            q = k // 8
            r = k % 8
            qv = jnp.full((16,), q, jnp.int32)
            sv = plsc.load_gather(rows_v.at[t],
                                  [k * 16 + iot, jnp.full((16,), 8, jnp.int32)])
            av = plsc.load_gather(adv_v.at[t], [qv, r * 16 + iot])
            sv = sv + av
            sv = jnp.where(sv > 0, sv, 0.2 * sv)
            w = jnp.exp(sv - cv)
            plsc.store_scatter(aw_v.at[t], [k * 16 + iot], w)

        @plsc.parallel_loop(0, CHUNK, unroll=2)
        def _me(e):
            ev = jnp.full((16,), e, jnp.int32)
            m = plsc.load_gather(aw_v.at[t], [ev])
            r = plsc.load_gather(rows_v.at[t], [ev, iot])
            plsc.store_scatter(rows_v.at[t], [ev, iot], r * m)

    def zinit(tt, c):
        b = tt * NSUB + sid

        @pl.when(b < NBLK)
        def _():
            pltpu.sync_copy(zb16, acc_sh.at[pl.ds(b * 128, 128)])
        return c

    lax.fori_loop(0, (NBLK + NSUB - 1) // NSUB, zinit, 0)
    fire_g(0, 0)
    plsc.subcore_barrier()

    def outer(i, c_):
        for b in range(NRING):
            cthis = i * NRING + b
            b2 = (b - 2) % NRING
            if b >= 2:
                drain_s(b2)
            else:
                @pl.when(i >= 1)
                def _():
                    drain_s(b2)
            if b == NRING - 1:
                @pl.when(cthis + 1 < CH_PER_SUB)
                def _():
                    fire_g(cthis + 1, 0)
            else:
                fire_g(cthis + 1, b + 1)
            drain_g(b)
            compute(b)
            fire_s(b)
        return c_

    lax.fori_loop(0, CH_PER_SUB // NRING, outer, 0)
    drain_s(2)
    drain_s(3)
    plsc.subcore_barrier()

    def cout(tt, c):
        b = tt * NSUB + sid

        @pl.when(b < NBLK)
        def _():
            pltpu.sync_copy(acc_sh.at[pl.ds(b * 128, 128)], zb16)
            pltpu.sync_copy(zb16, acc_o.at[cid].at[pl.ds(b * 128, 128)])
        return c

    lax.fori_loop(0, (NBLK + NSUB - 1) // NSUB, cout, 0)


def _k4(src2, dst2, h2t, adt2, cvech, z16):
    mesh = plsc.VectorSubcoreMesh(core_axis_name="c", subcore_axis_name="s")
    fn = pl.kernel(
        _k4_body,
        out_type=[
            jax.ShapeDtypeStruct((2, SP_ROWS, D2), jnp.float32),
        ],
        mesh=mesh,
        compiler_params=pltpu.CompilerParams(
            use_tc_tiling_on_sc=False, needs_layout_passes=False),
        scratch_types=[
            pltpu.VMEM((NRING, SUBQ, SUBC), jnp.int32),
            pltpu.VMEM((NRING, SUBQ, SUBC), jnp.int32),
            pltpu.VMEM((NRING, SUBQ, SUBC), jnp.int32),
            pltpu.VMEM((NRING, CHUNK, D2), jnp.float32),
            pltpu.VMEM((NRING, SUBQ, SUBC), jnp.float32),
            pltpu.VMEM((NRING, CHUNK), jnp.float32),
            pltpu.VMEM((128, D2), jnp.float32),
            pltpu.VMEM((16,), jnp.float32),
            pltpu.VMEM_SHARED((SP_ROWS, D2), jnp.float32),
            pltpu.SemaphoreType.DMA,
            pltpu.SemaphoreType.DMA,
            pltpu.SemaphoreType.DMA,
            pltpu.SemaphoreType.DMA,
            pltpu.SemaphoreType.DMA,
            pltpu.SemaphoreType.DMA,
            pltpu.SemaphoreType.DMA,
            pltpu.SemaphoreType.DMA,
        ],
    )
    return fn(src2, dst2, h2t, adt2, cvech, z16)


def _k5(acc2, b2):
    BN = 1024
    grid = (2 * SP_ROWS // BN,)

    def body(a_ref, b2_ref, o_ref):
        a = a_ref[...]
        v = a[:, 0:C2] / (a[:, C2:C2 + 1] + 1e-16) + b2_ref[...]
        m = jnp.max(v, axis=-1, keepdims=True)
        vs = v - m
        o_ref[...] = vs - jnp.log(jnp.sum(jnp.exp(vs), axis=-1, keepdims=True))

    return pl.pallas_call(
        body,
        grid=grid,
        in_specs=[
            pl.BlockSpec((BN, D2), lambda i: (i, 0)),
            pl.BlockSpec((1, C2), lambda i: (0, 0)),
        ],
        out_specs=pl.BlockSpec((BN, C2), lambda i: (i, 0)),
        out_shape=jax.ShapeDtypeStruct((2 * SP_ROWS, C2), jnp.float32),
    )(acc2, b2)


def _lrelu(v):
    return jnp.where(v > 0, v, 0.2 * v)


@jax.jit
def _run(x, edge_index, W1, att_src1, att_dst1, b1, W2, att_src2, att_dst2, b2):
    xs = x[0]
    ei = edge_index[0].astype(jnp.int32)
    src = jnp.concatenate([ei[0], jnp.zeros((EP - E,), jnp.int32)])
    dst = jnp.concatenate([ei[1], jnp.full((EP - E,), N, jnp.int32)])
    src2 = src.reshape(EROWS, SUBC)
    dst2 = dst.reshape(EROWS, SUBC)

    eye8 = jnp.eye(H1, dtype=jnp.float32)
    As = (att_src1[0][:, :, None] * eye8[:, None, :]).reshape(D1, H1)
    Ad = (att_dst1[0][:, :, None] * eye8[:, None, :]).reshape(D1, H1)

    h1, asr, adr = _k1(xs, W1, As, Ad)

    c1 = _lrelu(jnp.max(asr, axis=0) + jnp.max(adr, axis=0))
    cvec1 = jnp.tile(c1, 2)
    adrt = jnp.concatenate([adr, jnp.zeros((NPAD - N, H1), jnp.float32)])

    z64 = jnp.zeros((128, D1), jnp.float32)
    z8 = jnp.zeros((128, H1), jnp.float32)
    accs, dens = _k2(src2, dst2, h1, asr, adrt, cvec1, z64, z8)

    a_s = att_src2[0, 0]
    a_d = att_dst2[0, 0]
    w2p = jnp.zeros((D1, D2), jnp.float32)
    w2p = w2p.at[:, 0:C2].set(W2)
    w2p = w2p.at[:, 8].set(W2 @ a_s)
    w2p = w2p.at[:, 9].set(W2 @ a_d)
    qrow = jnp.zeros((1, D2), jnp.float32).at[0, C2].set(1.0)
    rrep = (eye8[:, :, None] * jnp.ones((1, 1, 8))).reshape(H1, D1)

    h2t, as2d, ad2d = _k3(accs.reshape(N3, D1), dens.reshape(N3, H1),
                          b1.reshape(1, D1), w2p, qrow, rrep)

    c2 = _lrelu(jnp.max(as2d) + jnp.max(ad2d))
    cvec2 = jnp.full((16,), c2, jnp.float32)
    adt2 = ad2d.reshape(N3)
    z16 = jnp.zeros((128, D2), jnp.float32)

    (acc2s,) = _k4(src2, dst2, h2t, adt2, cvec2, z16)

    o = _k5(acc2s.reshape(2 * SP_ROWS, D2), b2.reshape(1, C2))
    out = jnp.concatenate([o[:HALF], o[SP_ROWS:SP_ROWS + HALF]])
    return out.reshape(1, N, C2)


def kernel(x, edge_index, W1, att_src1, att_dst1, b1, W2, att_src2, att_dst2, b2):
    return _run(x, edge_index, W1, att_src1, att_dst1, b1,
                W2, att_src2, att_dst2, b2)
